# baseline (speedup 1.0000x reference)
import functools

import jax
import jax.numpy as jnp
from jax.experimental import pallas as pl


def _body(emb_ref, bbox_ref, kp_ref, vis_ref, maskflat_ref, mask_ref,
          w1e_ref, w1b_ref, w1k_ref, w1v_ref, b1_ref, w2_ref, b2_ref,
          out_ref, *, T, S, inv_s):
    bf = jnp.bfloat16
    f32 = jnp.float32
    h = jax.lax.dot_general(
        emb_ref[...].astype(bf), w1e_ref[...], (((1,), (1,)), ((), ())),
        preferred_element_type=f32)
    h += jax.lax.dot_general(
        kp_ref[...].astype(bf), w1k_ref[...], (((1,), (1,)), ((), ())),
        preferred_element_type=f32)
    h += jax.lax.dot_general(
        bbox_ref[...].astype(bf), w1b_ref[...], (((1,), (1,)), ((), ())),
        preferred_element_type=f32)
    h += vis_ref[...] * w1v_ref[...]
    h = jnp.maximum(h + b1_ref[...], 0.0)
    h *= maskflat_ref[...]
    hs = h.reshape(T, S, h.shape[-1]).sum(axis=1)
    cnt = mask_ref[...].sum(axis=1)
    out = jax.lax.dot_general(
        hs, w2_ref[...], (((1,), (1,)), ((), ())),
        preferred_element_type=f32)
    out_ref[...] = (out + cnt[:, None] * b2_ref[...]) * inv_s


def kernel(embeddings, visibility_scores, bbox_ltwh, keypoints_xyc,
           feats_masks, W1, b1, W2, b2):
    B, N, S, E = embeddings.shape
    KP = keypoints_xyc.shape[3]
    K3 = KP * 3
    M = B * N
    R = M * S
    F = W1.shape[1]
    TOK = W2.shape[0]

    T = 32

    emb = embeddings.reshape(R, E)
    vis = visibility_scores.reshape(R, 1)
    bbox = bbox_ltwh.reshape(R, 4)
    kp = keypoints_xyc.reshape(R, K3)
    maskf = feats_masks.reshape(M, S).astype(jnp.float32)
    maskflat = maskf.reshape(R, 1)

    bf = jnp.bfloat16
    W1e = W1[:, :E].astype(bf)
    W1v = W1[:, E].reshape(1, F)
    W1b = W1[:, E + 1:E + 5].astype(bf)
    W1k = W1[:, E + 5:].astype(bf)
    b1r = b1.reshape(1, F)
    b2r = b2.reshape(1, TOK)

    rows = T * S
    grid = (M // T,)
    body = functools.partial(_body, T=T, S=S, inv_s=1.0 / S)
    out = pl.pallas_call(
        body,
        grid=grid,
        in_specs=[
            pl.BlockSpec((rows, E), lambda i: (i, 0)),
            pl.BlockSpec((rows, 4), lambda i: (i, 0)),
            pl.BlockSpec((rows, K3), lambda i: (i, 0)),
            pl.BlockSpec((rows, 1), lambda i: (i, 0)),
            pl.BlockSpec((rows, 1), lambda i: (i, 0)),
            pl.BlockSpec((T, S), lambda i: (i, 0)),
            pl.BlockSpec((F, E), lambda i: (0, 0)),
            pl.BlockSpec((F, 4), lambda i: (0, 0)),
            pl.BlockSpec((F, K3), lambda i: (0, 0)),
            pl.BlockSpec((1, F), lambda i: (0, 0)),
            pl.BlockSpec((1, F), lambda i: (0, 0)),
            pl.BlockSpec((TOK, F), lambda i: (0, 0)),
            pl.BlockSpec((1, TOK), lambda i: (0, 0)),
        ],
        out_specs=pl.BlockSpec((T, TOK), lambda i: (i, 0)),
        out_shape=jax.ShapeDtypeStruct((M, TOK), jnp.float32),
    )(emb, bbox, kp, vis, maskflat, maskf, W1e, W1b, W1k, W1v, b1r, W2, b2r)
    return out.reshape(B, N, TOK)


# speedup vs baseline: 2.4299x; 2.4299x over previous
import functools

import jax
import jax.numpy as jnp
from jax.experimental import pallas as pl


def _body(emb_ref, vis_ref, bbox_ref, kp_ref, mask_ref,
          w1evb_ref, w1k_ref, b1_ref, w2a_ref, out_ref, *, T, S, inv_s):
    bf = jnp.bfloat16
    f32 = jnp.float32
    dn = (((1,), (0,)), ((), ()))
    w1evb = w1evb_ref[...]
    w1k = w1k_ref[...]
    b1c = b1_ref[...]
    m = mask_ref[...]
    ones_row = jnp.ones((1, S), dtype=bf)
    cols = []
    for t in range(T):
        evb = jnp.concatenate(
            [emb_ref[t], bbox_ref[t], vis_ref[t]], axis=0).astype(bf)
        kp_t = kp_ref[0, :, t * S:(t + 1) * S].astype(bf)
        h = jax.lax.dot_general(w1evb, evb, dn, preferred_element_type=f32)
        h += jax.lax.dot_general(w1k, kp_t, dn, preferred_element_type=f32)
        h = jnp.maximum(h + b1c, 0.0).astype(bf)
        haug = jnp.concatenate([h, ones_row], axis=0)
        mcol = m[t].reshape(S, 1).astype(bf)
        cols.append(jax.lax.dot_general(
            haug, mcol, dn, preferred_element_type=f32))
    hs = jnp.concatenate(cols, axis=1).astype(bf)
    out = jax.lax.dot_general(
        w2a_ref[...], hs, dn, preferred_element_type=f32)
    out_ref[...] = out[None] * inv_s


def kernel(embeddings, visibility_scores, bbox_ltwh, keypoints_xyc,
           feats_masks, W1, b1, W2, b2):
    B, N, S, E = embeddings.shape
    KP = keypoints_xyc.shape[3]
    K3 = KP * 3
    M = B * N
    F = W1.shape[1]
    O = W2.shape[0]

    T = 32
    NB = N // T

    embT = embeddings.transpose(0, 1, 3, 2).reshape(M, E, S)
    visT = visibility_scores.transpose(0, 1, 3, 2).reshape(M, 1, S)
    bboxT = bbox_ltwh.transpose(0, 1, 3, 2).reshape(M, 4, S)
    kpT = keypoints_xyc.transpose(0, 3, 4, 1, 2).reshape(B, K3, N * S)
    maskf = feats_masks.astype(jnp.float32).reshape(M, S)

    bf = jnp.bfloat16
    W1evb = jnp.concatenate(
        [W1[:, :E], W1[:, E + 1:E + 5], W1[:, E:E + 1]], axis=1).astype(bf)
    W1k = W1[:, E + 5:].astype(bf)
    b1col = b1.reshape(F, 1)
    W2aug = jnp.concatenate([W2, b2[:, None]], axis=1).astype(bf)

    grid = (M // T,)
    body = functools.partial(_body, T=T, S=S, inv_s=1.0 / S)
    out = pl.pallas_call(
        body,
        grid=grid,
        in_specs=[
            pl.BlockSpec((T, E, S), lambda i: (i, 0, 0)),
            pl.BlockSpec((T, 1, S), lambda i: (i, 0, 0)),
            pl.BlockSpec((T, 4, S), lambda i: (i, 0, 0)),
            pl.BlockSpec((1, K3, T * S), lambda i: (i // NB, 0, i % NB)),
            pl.BlockSpec((T, S), lambda i: (i, 0)),
            pl.BlockSpec((F, E + 5), lambda i: (0, 0)),
            pl.BlockSpec((F, K3), lambda i: (0, 0)),
            pl.BlockSpec((F, 1), lambda i: (0, 0)),
            pl.BlockSpec((O, F + 1), lambda i: (0, 0)),
        ],
        out_specs=pl.BlockSpec((1, O, T), lambda i: (i, 0, 0)),
        out_shape=jax.ShapeDtypeStruct((M // T, O, T), jnp.float32),
    )(embT, visT, bboxT, kpT, maskf, W1evb, W1k, b1col, W2aug)
    return out.transpose(0, 2, 1).reshape(B, N, O)


# speedup vs baseline: 2.5107x; 1.0332x over previous
import functools

import jax
import jax.numpy as jnp
from jax.experimental import pallas as pl


def _body(emb_ref, vis_ref, bbox_ref, kp_ref, mask_ref,
          w1evb_ref, w1k_ref, w2a_ref, out_ref, *, T, S, inv_s):
    bf = jnp.bfloat16
    f32 = jnp.float32
    dn = (((1,), (0,)), ((), ()))
    w1evb = w1evb_ref[...]
    w1k = w1k_ref[...]
    m = mask_ref[...]
    ones_row = jnp.ones((1, S), dtype=bf)
    ones_f32 = jnp.ones((1, S), dtype=f32)
    cols = []
    for t in range(T):
        evb = jnp.concatenate(
            [emb_ref[t], bbox_ref[t], vis_ref[t], ones_f32],
            axis=0).astype(bf)
        kp_t = kp_ref[0, :, t * S:(t + 1) * S].astype(bf)
        h = jax.lax.dot_general(w1evb, evb, dn, preferred_element_type=f32)
        h += jax.lax.dot_general(w1k, kp_t, dn, preferred_element_type=f32)
        h = jnp.maximum(h, 0.0).astype(bf)
        haug = jnp.concatenate([h, ones_row], axis=0)
        mcol = m[t].reshape(S, 1).astype(bf)
        cols.append(jax.lax.dot_general(
            haug, mcol, dn, preferred_element_type=f32))
    hs = jnp.concatenate(cols, axis=1).astype(bf)
    out = jax.lax.dot_general(
        w2a_ref[...], hs, dn, preferred_element_type=f32)
    out_ref[...] = out[None] * inv_s


def kernel(embeddings, visibility_scores, bbox_ltwh, keypoints_xyc,
           feats_masks, W1, b1, W2, b2):
    B, N, S, E = embeddings.shape
    KP = keypoints_xyc.shape[3]
    K3 = KP * 3
    M = B * N
    F = W1.shape[1]
    O = W2.shape[0]

    T = 64
    NB = N // T

    embT = embeddings.transpose(0, 1, 3, 2).reshape(M, E, S)
    visT = visibility_scores.transpose(0, 1, 3, 2).reshape(M, 1, S)
    bboxT = bbox_ltwh.transpose(0, 1, 3, 2).reshape(M, 4, S)
    kpT = keypoints_xyc.transpose(0, 3, 4, 1, 2).reshape(B, K3, N * S)
    maskf = feats_masks.astype(jnp.float32).reshape(M, S)

    bf = jnp.bfloat16
    W1evb = jnp.concatenate(
        [W1[:, :E], W1[:, E + 1:E + 5], W1[:, E:E + 1], b1[:, None]],
        axis=1).astype(bf)
    W1k = W1[:, E + 5:].astype(bf)
    W2aug = jnp.concatenate([W2, b2[:, None]], axis=1).astype(bf)

    grid = (M // T,)
    body = functools.partial(_body, T=T, S=S, inv_s=1.0 / S)
    out = pl.pallas_call(
        body,
        grid=grid,
        in_specs=[
            pl.BlockSpec((T, E, S), lambda i: (i, 0, 0)),
            pl.BlockSpec((T, 1, S), lambda i: (i, 0, 0)),
            pl.BlockSpec((T, 4, S), lambda i: (i, 0, 0)),
            pl.BlockSpec((1, K3, T * S), lambda i: (i // NB, 0, i % NB)),
            pl.BlockSpec((T, S), lambda i: (i, 0)),
            pl.BlockSpec((F, E + 6), lambda i: (0, 0)),
            pl.BlockSpec((F, K3), lambda i: (0, 0)),
            pl.BlockSpec((O, F + 1), lambda i: (0, 0)),
        ],
        out_specs=pl.BlockSpec((1, O, T), lambda i: (i, 0, 0)),
        out_shape=jax.ShapeDtypeStruct((M // T, O, T), jnp.float32),
    )(embT, visT, bboxT, kpT, maskf, W1evb, W1k, W2aug)
    return out.transpose(0, 2, 1).reshape(B, N, O)


# speedup vs baseline: 4.4416x; 1.7691x over previous
import functools

import jax
import jax.numpy as jnp
from jax.experimental import pallas as pl


def _body(emb_ref, vis_ref, bbox_ref, kp_ref, mask_ref,
          w1evb_ref, w1k_ref, w2a_ref, out_ref, *, T, S, inv_s):
    bf = jnp.bfloat16
    f32 = jnp.float32
    dn = (((1,), (0,)), ((), ()))
    m = mask_ref[...]
    TS = T * S
    EMB = jnp.concatenate([emb_ref[t] for t in range(T)], axis=1)
    BBX = jnp.concatenate([bbox_ref[t] for t in range(T)], axis=1)
    VIS = jnp.concatenate([vis_ref[t] for t in range(T)], axis=1)
    EVB = jnp.concatenate(
        [EMB, BBX, VIS, jnp.ones((1, TS), dtype=f32)], axis=0).astype(bf)
    h = jax.lax.dot_general(
        w1evb_ref[...], EVB, dn, preferred_element_type=f32)
    h += jax.lax.dot_general(
        w1k_ref[...], kp_ref[0].astype(bf), dn, preferred_element_type=f32)
    h = jnp.maximum(h, 0.0).astype(bf)
    haug = jnp.concatenate(
        [h, jnp.ones((1, TS), dtype=bf)], axis=0)
    cols = []
    for t in range(T):
        mcol = m[t].reshape(S, 1).astype(bf)
        cols.append(jax.lax.dot_general(
            haug[:, t * S:(t + 1) * S], mcol, dn,
            preferred_element_type=f32))
    hs = jnp.concatenate(cols, axis=1).astype(bf)
    out = jax.lax.dot_general(
        w2a_ref[...], hs, dn, preferred_element_type=f32)
    out_ref[...] = out[None] * inv_s


def kernel(embeddings, visibility_scores, bbox_ltwh, keypoints_xyc,
           feats_masks, W1, b1, W2, b2):
    B, N, S, E = embeddings.shape
    KP = keypoints_xyc.shape[3]
    K3 = KP * 3
    M = B * N
    F = W1.shape[1]
    O = W2.shape[0]

    T = 64
    NB = N // T

    embT = embeddings.transpose(0, 1, 3, 2).reshape(M, E, S)
    visT = visibility_scores.transpose(0, 1, 3, 2).reshape(M, 1, S)
    bboxT = bbox_ltwh.transpose(0, 1, 3, 2).reshape(M, 4, S)
    kpT = keypoints_xyc.transpose(0, 3, 4, 1, 2).reshape(B, K3, N * S)
    maskf = feats_masks.astype(jnp.float32).reshape(M, S)

    bf = jnp.bfloat16
    W1evb = jnp.concatenate(
        [W1[:, :E], W1[:, E + 1:E + 5], W1[:, E:E + 1], b1[:, None]],
        axis=1).astype(bf)
    W1k = W1[:, E + 5:].astype(bf)
    W2aug = jnp.concatenate([W2, b2[:, None]], axis=1).astype(bf)

    grid = (M // T,)
    body = functools.partial(_body, T=T, S=S, inv_s=1.0 / S)
    out = pl.pallas_call(
        body,
        grid=grid,
        in_specs=[
            pl.BlockSpec((T, E, S), lambda i: (i, 0, 0)),
            pl.BlockSpec((T, 1, S), lambda i: (i, 0, 0)),
            pl.BlockSpec((T, 4, S), lambda i: (i, 0, 0)),
            pl.BlockSpec((1, K3, T * S), lambda i: (i // NB, 0, i % NB)),
            pl.BlockSpec((T, S), lambda i: (i, 0)),
            pl.BlockSpec((F, E + 6), lambda i: (0, 0)),
            pl.BlockSpec((F, K3), lambda i: (0, 0)),
            pl.BlockSpec((O, F + 1), lambda i: (0, 0)),
        ],
        out_specs=pl.BlockSpec((1, O, T), lambda i: (i, 0, 0)),
        out_shape=jax.ShapeDtypeStruct((M // T, O, T), jnp.float32),
    )(embT, visT, bboxT, kpT, maskf, W1evb, W1k, W2aug)
    return out.transpose(0, 2, 1).reshape(B, N, O)


# speedup vs baseline: 4.9325x; 1.1105x over previous
import functools

import jax
import jax.numpy as jnp
from jax.experimental import pallas as pl


def _body(emb_ref, vis_ref, bbox_ref, kp_ref, mask_ref,
          w1evb_ref, w1k_ref, w2a_ref, sel_ref, out_ref, *, T, S, inv_s):
    bf = jnp.bfloat16
    f32 = jnp.float32
    dn = (((1,), (0,)), ((), ()))
    m = mask_ref[...]
    TS = T * S
    EMB = jnp.concatenate([emb_ref[t] for t in range(T)], axis=1)
    BBX = jnp.concatenate([bbox_ref[t] for t in range(T)], axis=1)
    VIS = jnp.concatenate([vis_ref[t] for t in range(T)], axis=1)
    EVB = jnp.concatenate(
        [EMB, BBX, VIS, jnp.ones((1, TS), dtype=f32)], axis=0).astype(bf)
    h = jax.lax.dot_general(
        w1evb_ref[...], EVB, dn, preferred_element_type=f32)
    h += jax.lax.dot_general(
        w1k_ref[...], kp_ref[0].astype(bf), dn, preferred_element_type=f32)
    h = jnp.maximum(h, 0.0).astype(bf)
    mrow = jnp.concatenate(
        [m[t:t + 1, :] for t in range(T)], axis=1).astype(bf)
    haug = jnp.concatenate([h * mrow, mrow], axis=0)
    hs = jax.lax.dot_general(
        haug, sel_ref[...], dn, preferred_element_type=f32)
    out = jax.lax.dot_general(
        w2a_ref[...], hs.astype(bf), dn,
        preferred_element_type=f32)
    out_ref[...] = out[None] * inv_s


def kernel(embeddings, visibility_scores, bbox_ltwh, keypoints_xyc,
           feats_masks, W1, b1, W2, b2):
    B, N, S, E = embeddings.shape
    KP = keypoints_xyc.shape[3]
    K3 = KP * 3
    M = B * N
    F = W1.shape[1]
    O = W2.shape[0]

    T = 64
    NB = N // T

    embT = embeddings.transpose(0, 1, 3, 2).reshape(M, E, S)
    visT = visibility_scores.transpose(0, 1, 3, 2).reshape(M, 1, S)
    bboxT = bbox_ltwh.transpose(0, 1, 3, 2).reshape(M, 4, S)
    kpT = keypoints_xyc.transpose(0, 3, 4, 1, 2).reshape(B, K3, N * S)
    maskf = feats_masks.astype(jnp.float32).reshape(M, S)

    bf = jnp.bfloat16
    W1evb = jnp.concatenate(
        [W1[:, :E], W1[:, E + 1:E + 5], W1[:, E:E + 1], b1[:, None]],
        axis=1).astype(bf)
    W1k = W1[:, E + 5:].astype(bf)
    W2aug = jnp.concatenate([W2, b2[:, None]], axis=1).astype(bf)
    sel01 = (jnp.arange(T * S)[:, None] // S
             == jnp.arange(T)[None, :]).astype(bf)

    grid = (M // T,)
    body = functools.partial(_body, T=T, S=S, inv_s=1.0 / S)
    out = pl.pallas_call(
        body,
        grid=grid,
        in_specs=[
            pl.BlockSpec((T, E, S), lambda i: (i, 0, 0)),
            pl.BlockSpec((T, 1, S), lambda i: (i, 0, 0)),
            pl.BlockSpec((T, 4, S), lambda i: (i, 0, 0)),
            pl.BlockSpec((1, K3, T * S), lambda i: (i // NB, 0, i % NB)),
            pl.BlockSpec((T, S), lambda i: (i, 0)),
            pl.BlockSpec((F, E + 6), lambda i: (0, 0)),
            pl.BlockSpec((F, K3), lambda i: (0, 0)),
            pl.BlockSpec((O, F + 1), lambda i: (0, 0)),
            pl.BlockSpec((T * S, T), lambda i: (0, 0)),
        ],
        out_specs=pl.BlockSpec((1, O, T), lambda i: (i, 0, 0)),
        out_shape=jax.ShapeDtypeStruct((M // T, O, T), jnp.float32),
    )(embT, visT, bboxT, kpT, maskf, W1evb, W1k, W2aug, sel01)
    return out.transpose(0, 2, 1).reshape(B, N, O)


# speedup vs baseline: 5.1499x; 1.0441x over previous
import functools

import jax
import jax.numpy as jnp
from jax.experimental import pallas as pl


def _body(emb_ref, vis_ref, bbox_ref, kp_ref, mask_ref,
          w1evb_ref, w1k_ref, w2a_ref, sel_ref, out_ref, *, T, S, inv_s):
    bf = jnp.bfloat16
    f32 = jnp.float32
    dn = (((1,), (0,)), ((), ()))
    m = mask_ref[...]
    TS = T * S
    EMB = jnp.concatenate([emb_ref[t] for t in range(T)], axis=1)
    BBX = jnp.concatenate([bbox_ref[t] for t in range(T)], axis=1)
    VIS = jnp.concatenate([vis_ref[t] for t in range(T)], axis=1)
    EVB = jnp.concatenate(
        [EMB, BBX, VIS, jnp.ones((1, TS), dtype=f32)], axis=0).astype(bf)
    h = jax.lax.dot_general(
        w1evb_ref[...], EVB, dn, preferred_element_type=f32)
    h += jax.lax.dot_general(
        w1k_ref[...], kp_ref[0].astype(bf), dn, preferred_element_type=f32)
    h = jnp.maximum(h, 0.0).astype(bf)
    mrow = jnp.concatenate(
        [m[t:t + 1, :] for t in range(T)], axis=1).astype(bf)
    haug = jnp.concatenate([h * mrow, mrow], axis=0)
    hs = jax.lax.dot_general(
        haug, sel_ref[...], dn, preferred_element_type=f32)
    out = jax.lax.dot_general(
        w2a_ref[...], hs.astype(bf), dn,
        preferred_element_type=f32)
    out_ref[...] = out[None] * inv_s


def kernel(embeddings, visibility_scores, bbox_ltwh, keypoints_xyc,
           feats_masks, W1, b1, W2, b2):
    B, N, S, E = embeddings.shape
    KP = keypoints_xyc.shape[3]
    K3 = KP * 3
    M = B * N
    F = W1.shape[1]
    O = W2.shape[0]

    T = 128
    NB = N // T

    embT = embeddings.transpose(0, 1, 3, 2).reshape(M, E, S)
    visT = visibility_scores.transpose(0, 1, 3, 2).reshape(M, 1, S)
    bboxT = bbox_ltwh.transpose(0, 1, 3, 2).reshape(M, 4, S)
    kpT = keypoints_xyc.transpose(0, 3, 4, 1, 2).reshape(B, K3, N * S)
    maskf = feats_masks.astype(jnp.float32).reshape(M, S)

    bf = jnp.bfloat16
    W1evb = jnp.concatenate(
        [W1[:, :E], W1[:, E + 1:E + 5], W1[:, E:E + 1], b1[:, None]],
        axis=1).astype(bf)
    W1k = W1[:, E + 5:].astype(bf)
    W2aug = jnp.concatenate([W2, b2[:, None]], axis=1).astype(bf)
    sel01 = (jnp.arange(T * S)[:, None] // S
             == jnp.arange(T)[None, :]).astype(bf)

    grid = (M // T,)
    body = functools.partial(_body, T=T, S=S, inv_s=1.0 / S)
    out = pl.pallas_call(
        body,
        grid=grid,
        in_specs=[
            pl.BlockSpec((T, E, S), lambda i: (i, 0, 0)),
            pl.BlockSpec((T, 1, S), lambda i: (i, 0, 0)),
            pl.BlockSpec((T, 4, S), lambda i: (i, 0, 0)),
            pl.BlockSpec((1, K3, T * S), lambda i: (i // NB, 0, i % NB)),
            pl.BlockSpec((T, S), lambda i: (i, 0)),
            pl.BlockSpec((F, E + 6), lambda i: (0, 0)),
            pl.BlockSpec((F, K3), lambda i: (0, 0)),
            pl.BlockSpec((O, F + 1), lambda i: (0, 0)),
            pl.BlockSpec((T * S, T), lambda i: (0, 0)),
        ],
        out_specs=pl.BlockSpec((1, O, T), lambda i: (i, 0, 0)),
        out_shape=jax.ShapeDtypeStruct((M // T, O, T), jnp.float32),
    )(embT, visT, bboxT, kpT, maskf, W1evb, W1k, W2aug, sel01)
    return out.transpose(0, 2, 1).reshape(B, N, O)
